# baseline (speedup 1.0000x reference)
import functools

import jax
import jax.numpy as jnp
from jax.experimental import pallas as pl
from jax.experimental.pallas import tpu as pltpu

B, H, DIM, RANK = 64, 8, 256, 64
EPS = 1e-06
QROWS = 4 * RANK
HALF = DIM // 2


def _to_q(x):
    return (x.reshape(B, H, 2, HALF, RANK, 2)
             .transpose(0, 1, 4, 2, 5, 3)
             .reshape(B, H, QROWS, HALF))


def _from_q(xq):
    return (xq.reshape(B, H, RANK, 2, 2, HALF)
              .transpose(0, 1, 3, 5, 2, 4)
              .reshape(B, H, DIM, RANK, 2))


def _to_q_vec(x):
    return (x.reshape(B, H, 2, HALF, 2)
             .transpose(0, 1, 2, 4, 3)
             .reshape(B, H, 4, HALF))


def _from_q_vec(xq):
    return (xq.reshape(B, H, 2, 2, HALF)
              .transpose(0, 1, 2, 4, 3)
              .reshape(B, H, DIM, 2))


def _sasaki_kernel(ns_ref, gamma_ref, u_ref, v_ref, kq_ref, vq_ref,
                   yq_ref, uo_ref, vo_ref):
    f32 = jnp.float32
    NB = 4
    NH = NB * H
    bi0 = pl.program_id(0) * NB

    si4 = jax.lax.broadcasted_iota(jnp.int32, (QROWS, 1), 0) // 4
    liq = jax.lax.broadcasted_iota(jnp.int32, (1, QROWS), 1) % 4
    e0 = (liq == 0).astype(f32)
    e1 = (liq == 1).astype(f32)
    e2 = (liq == 2).astype(f32)
    e3 = (liq == 3).astype(f32)
    t4r = jax.lax.broadcasted_iota(jnp.int32, (QROWS, 4), 0) % 4
    t4c = jax.lax.broadcasted_iota(jnp.int32, (QROWS, 4), 1)
    T4 = (t4r == t4c).astype(f32)
    s4r = jax.lax.broadcasted_iota(jnp.int32, (QROWS, QROWS), 0) // 4
    s4c = jax.lax.broadcasted_iota(jnp.int32, (QROWS, QROWS), 1) // 4
    S4 = (s4r == s4c).astype(f32)
    obr = jax.lax.broadcasted_iota(jnp.int32, (2 * NH * HALF, 2 * NH), 0) // HALF
    obc = jax.lax.broadcasted_iota(jnp.int32, (2 * NH * HALF, 2 * NH), 1)
    OBD = (obr == obc).astype(f32)

    def mm(a, b, precision=None):
        return jax.lax.dot_general(a, b, (((1,), (0,)), ((), ())),
                                   preferred_element_type=f32,
                                   precision=precision)

    def wrows(crT, ciT):
        w0 = crT * e0 - ciT * e1
        w1 = ciT * e0 + crT * e1
        w2 = crT * e2 - ciT * e3
        w3 = ciT * e2 + crT * e3
        return jnp.concatenate([w0, w1, w2, w3], axis=0)

    A = [u_ref[b, h] for b in range(NB) for h in range(H)]
    kq = [kq_ref[b, h] for b in range(NB) for h in range(H)]
    kqs = [jnp.concatenate([k[1:2], -k[0:1], k[3:4], -k[2:3]], axis=0)
           for k in kq]
    j = [ns_ref[bi0 + b, h] for b in range(NB) for h in range(H)]
    rowm = [si4 == jh for jh in j]

    G1 = mm(T4, jnp.concatenate(kq, axis=1))
    G2 = mm(T4, jnp.concatenate(kqs, axis=1))
    TMP = jnp.concatenate(
        [jnp.concatenate(A, axis=1) * G1,
         jnp.concatenate(A, axis=1) * G2], axis=1)

    rdcat = mm(TMP, OBD)
    cc = mm(S4, rdcat)
    ct = jax.lax.transpose(cc, (1, 0))

    Wm = [wrows(ct[n:n + 1], ct[NH + n:NH + n + 1]) for n in range(NH)]
    KP = [mm(Wm[n], A[n]) for n in range(NH)]
    k_perp = [kq[n] - KP[n] for n in range(NH)]
    inv = [jax.lax.rsqrt(jnp.maximum(jnp.sum(kp * kp), EPS * EPS))
           for kp in k_perp]
    u_new = [k_perp[n] * inv[n] for n in range(NH)]

    dr = [jnp.sum(u_new[n] * kq[n]) for n in range(NH)]
    di = [jnp.sum(u_new[n] * kqs[n]) for n in range(NH)]
    ccq = jnp.concatenate(
        [jnp.where(rowm[n], dr[n], cc[:, n:n + 1]) for n in range(NH)]
        + [jnp.where(rowm[n], di[n], cc[:, NH + n:NH + n + 1]) for n in range(NH)],
        axis=1)
    cqt = jax.lax.transpose(ccq, (1, 0))

    u_t = mm(T4, jnp.concatenate(u_new, axis=1))
    v_t = mm(T4, jnp.concatenate(
        [vq_ref[b, h] for b in range(NB) for h in range(H)], axis=1))

    for b in range(NB):
        for h in range(H):
            n = b * H + h
            g = jnp.clip(gamma_ref[bi0 + b, h], 0.0, 1.0)
            uo_ref[b, h] = jnp.where(rowm[n], u_t[:, n * HALF:(n + 1) * HALF],
                                     A[n])
            V_new = g * jnp.where(rowm[n], v_t[:, n * HALF:(n + 1) * HALF],
                                  v_ref[b, h])
            vo_ref[b, h] = V_new
            yq_ref[b, h] = mm(wrows(cqt[n:n + 1], cqt[NH + n:NH + n + 1]),
                              V_new)


@functools.partial(jax.jit, static_argnames=())
def kernel(U, V, k, v, gamma, next_slot, filled):
    Uq = _to_q(U)
    Vq = _to_q(V)
    kq = _to_q_vec(k)
    vq = _to_q_vec(v)

    big_spec = pl.BlockSpec((4, H, QROWS, HALF), lambda b: (b, 0, 0, 0))
    vec_spec = pl.BlockSpec((4, H, 4, HALF), lambda b: (b, 0, 0, 0))
    smem_spec = pl.BlockSpec(memory_space=pltpu.SMEM)

    yq, U_newq, V_newq = pl.pallas_call(
        _sasaki_kernel,
        grid=(B // 4,),
        in_specs=[smem_spec, smem_spec, big_spec, big_spec, vec_spec, vec_spec],
        out_specs=[vec_spec, big_spec, big_spec],
        out_shape=[
            jax.ShapeDtypeStruct((B, H, 4, HALF), jnp.float32),
            jax.ShapeDtypeStruct((B, H, QROWS, HALF), jnp.float32),
            jax.ShapeDtypeStruct((B, H, QROWS, HALF), jnp.float32),
        ],
    )(next_slot, gamma, Uq, Vq, kq, vq)

    y = _from_q_vec(yq)
    U_new = _from_q(U_newq)
    V_new = _from_q(V_newq)
    next_slot_new = (next_slot + 1) % RANK
    filled_new = jnp.minimum(filled + 1, jnp.full_like(filled, RANK))
    return (y, U_new, V_new, next_slot_new, filled_new)
